# baseline (speedup 1.0000x reference)
import functools

import jax
import jax.numpy as jnp
from jax import lax
from jax.experimental import pallas as pl
from jax.experimental.pallas import tpu as pltpu
from jax.experimental.pallas import tpu_sc as plsc

NW = 32
PADF = 128
VROW = 104
GRP = 64
LANES = 16

_GDN = lax.GatherDimensionNumbers(
    offset_dims=(), collapsed_slice_dims=(0,), start_index_map=(0,)
)


def _permute(v, idx):
    return lax.gather(
        v, idx[:, None], _GDN, slice_sizes=(1,),
        mode=lax.GatherScatterMode.PROMISE_IN_BOUNDS,
    )


def _lr_kernel(fids_hbm, table_hbm, out_hbm, idx_v, vals_v, out_v, sem):
    rows_per_w = out_v.shape[0]
    f_per_row = 100

    wid = lax.axis_index("s") * 2 + lax.axis_index("c")
    base = wid * rows_per_w

    pltpu.sync_copy(fids_hbm.at[pl.ds(base, rows_per_w), :], idx_v)

    iota = lax.iota(jnp.int32, LANES)
    perms = [iota ^ s for s in (8, 4, 2, 1)]
    tailmask = (iota >= 8) & (iota < 12)
    zeros16 = jnp.zeros((LANES,), jnp.float32)

    def fire(g):
        for b in range(GRP):
            j = g * GRP + b
            pltpu.async_copy(
                table_hbm.at[idx_v.at[j, pl.ds(0, f_per_row)]],
                vals_v.at[pl.ds(j * VROW, f_per_row)],
                sem,
            )

    def drain(g):
        for b in range(GRP):
            j = g * GRP + b
            pltpu.make_async_copy(
                table_hbm.at[idx_v.at[j, pl.ds(0, f_per_row)]],
                vals_v.at[pl.ds(j * VROW, f_per_row)],
                sem,
            ).wait()

    def reduce_grp(g):
        for blk in range(GRP // LANES):
            acc = zeros16
            for r in range(LANES):
                j = g * GRP + blk * LANES + r
                s = vals_v[pl.ds(j * VROW, LANES)]
                for k in range(1, 6):
                    s = s + vals_v[pl.ds(j * VROW + k * LANES, LANES)]
                tail = vals_v[pl.ds(j * VROW + 88, LANES)]
                s = s + jnp.where(tailmask, tail, 0.0)
                for p in perms:
                    s = s + _permute(s, p)
                acc = jnp.where(iota == r, s, acc)
            out_v[pl.ds(g * GRP + blk * LANES, LANES)] = acc

    n_grp = rows_per_w // GRP
    fire(jnp.int32(0))
    fire(jnp.int32(1))

    def pipe(g, _):
        fire(g)
        drain(g - 2)
        reduce_grp(g - 2)
        return 0

    lax.fori_loop(2, n_grp, pipe, 0)
    for g in (n_grp - 2, n_grp - 1):
        drain(jnp.int32(g))
        reduce_grp(jnp.int32(g))

    pltpu.sync_copy(out_v, out_hbm.at[pl.ds(base, rows_per_w)])


def kernel(fids, table):
    B, F = fids.shape
    vocab = table.shape[0]
    rows_per_w = B // NW
    assert B == NW * rows_per_w and F == 100 and rows_per_w % GRP == 0

    fids_p = jnp.pad(fids, ((0, 0), (0, PADF - F)))
    table_flat = table.reshape(vocab)

    mesh = plsc.VectorSubcoreMesh(core_axis_name="c", subcore_axis_name="s")
    run = functools.partial(
        pl.kernel,
        out_type=jax.ShapeDtypeStruct((B,), jnp.float32),
        mesh=mesh,
        scratch_types=[
            pltpu.VMEM((rows_per_w, PADF), jnp.int32),
            pltpu.VMEM((rows_per_w * VROW,), jnp.float32),
            pltpu.VMEM((rows_per_w,), jnp.float32),
            pltpu.SemaphoreType.DMA,
        ],
    )(_lr_kernel)
    return run(fids_p, table_flat)


# speedup vs baseline: 1.0138x; 1.0138x over previous
import functools

import jax
import jax.numpy as jnp
from jax import lax
from jax.experimental import pallas as pl
from jax.experimental.pallas import tpu as pltpu
from jax.experimental.pallas import tpu_sc as plsc

NW = 32
PADF = 128
VROW = 104
GRP = 32
LANES = 16

_GDN = lax.GatherDimensionNumbers(
    offset_dims=(), collapsed_slice_dims=(0,), start_index_map=(0,)
)


def _permute(v, idx):
    return lax.gather(
        v, idx[:, None], _GDN, slice_sizes=(1,),
        mode=lax.GatherScatterMode.PROMISE_IN_BOUNDS,
    )


def _lr_kernel(fids_hbm, table_hbm, out_hbm, idx_v, vals_v, out_v, sem):
    rows_per_w = out_v.shape[0]
    f_per_row = 100

    wid = lax.axis_index("s") * 2 + lax.axis_index("c")
    base = wid * rows_per_w

    pltpu.sync_copy(fids_hbm.at[pl.ds(base, rows_per_w), :], idx_v)

    iota = lax.iota(jnp.int32, LANES)
    perms = [iota ^ s for s in (8, 4, 2, 1)]
    tailmask = (iota >= 8) & (iota < 12)
    zeros16 = jnp.zeros((LANES,), jnp.float32)

    def fire(g):
        for b in range(GRP):
            j = g * GRP + b
            pltpu.async_copy(
                table_hbm.at[idx_v.at[j, pl.ds(0, f_per_row)]],
                vals_v.at[pl.ds(j * VROW, f_per_row)],
                sem,
            )

    def drain(g):
        for b in range(GRP):
            j = g * GRP + b
            pltpu.make_async_copy(
                table_hbm.at[idx_v.at[j, pl.ds(0, f_per_row)]],
                vals_v.at[pl.ds(j * VROW, f_per_row)],
                sem,
            ).wait()

    def reduce_grp(g):
        for blk in range(GRP // LANES):
            acc = zeros16
            for r in range(LANES):
                j = g * GRP + blk * LANES + r
                s = vals_v[pl.ds(j * VROW, LANES)]
                for k in range(1, 6):
                    s = s + vals_v[pl.ds(j * VROW + k * LANES, LANES)]
                tail = vals_v[pl.ds(j * VROW + 88, LANES)]
                s = s + jnp.where(tailmask, tail, 0.0)
                for p in perms:
                    s = s + _permute(s, p)
                acc = jnp.where(iota == r, s, acc)
            out_v[pl.ds(g * GRP + blk * LANES, LANES)] = acc

    n_grp = rows_per_w // GRP
    fire(jnp.int32(0))
    fire(jnp.int32(1))

    def pipe(g, _):
        fire(g)
        drain(g - 2)
        reduce_grp(g - 2)
        return 0

    lax.fori_loop(2, n_grp, pipe, 0)
    for g in (n_grp - 2, n_grp - 1):
        drain(jnp.int32(g))
        reduce_grp(jnp.int32(g))

    pltpu.sync_copy(out_v, out_hbm.at[pl.ds(base, rows_per_w)])


def kernel(fids, table):
    B, F = fids.shape
    vocab = table.shape[0]
    rows_per_w = B // NW
    assert B == NW * rows_per_w and F == 100 and rows_per_w % GRP == 0

    fids_p = jnp.pad(fids, ((0, 0), (0, PADF - F)))
    table_flat = table.reshape(vocab)

    mesh = plsc.VectorSubcoreMesh(core_axis_name="c", subcore_axis_name="s")
    run = functools.partial(
        pl.kernel,
        out_type=jax.ShapeDtypeStruct((B,), jnp.float32),
        mesh=mesh,
        scratch_types=[
            pltpu.VMEM((rows_per_w, PADF), jnp.int32),
            pltpu.VMEM((rows_per_w * VROW,), jnp.float32),
            pltpu.VMEM((rows_per_w,), jnp.float32),
            pltpu.SemaphoreType.DMA,
        ],
    )(_lr_kernel)
    return run(fids_p, table_flat)


# speedup vs baseline: 1.0168x; 1.0030x over previous
import functools

import jax
import jax.numpy as jnp
from jax import lax
from jax.experimental import pallas as pl
from jax.experimental.pallas import tpu as pltpu
from jax.experimental.pallas import tpu_sc as plsc

NW = 32
PADF = 128
VROW = 104
GRP = 32
LANES = 16

_GDN = lax.GatherDimensionNumbers(
    offset_dims=(), collapsed_slice_dims=(0,), start_index_map=(0,)
)


def _permute(v, idx):
    return lax.gather(
        v, idx[:, None], _GDN, slice_sizes=(1,),
        mode=lax.GatherScatterMode.PROMISE_IN_BOUNDS,
    )


def _lr_kernel(fids_hbm, table_hbm, out_hbm, idx_v, vals_v, out_v, sem):
    rows_per_w = out_v.shape[0]
    f_per_row = 100

    wid = lax.axis_index("s") * 2 + lax.axis_index("c")
    base = wid * rows_per_w

    pltpu.sync_copy(fids_hbm.at[pl.ds(base, rows_per_w), :], idx_v)

    iota = lax.iota(jnp.int32, LANES)
    perms = [iota ^ s for s in (8, 4, 2, 1)]
    tailmask = (iota >= 8) & (iota < 12)
    zeros16 = jnp.zeros((LANES,), jnp.float32)

    def fire(g):
        for b in range(GRP):
            j = g * GRP + b
            pltpu.async_copy(
                table_hbm.at[idx_v.at[j, pl.ds(0, f_per_row)]],
                vals_v.at[pl.ds(j * VROW, f_per_row)],
                sem,
            )

    def drain(g):
        pltpu.make_async_copy(
            table_hbm.at[pl.ds(0, GRP * f_per_row)],
            vals_v.at[pl.ds(g * GRP * VROW, GRP * f_per_row)],
            sem,
        ).wait()

    def reduce_grp(g):
        for blk in range(GRP // LANES):
            acc = zeros16
            for r in range(LANES):
                j = g * GRP + blk * LANES + r
                s = vals_v[pl.ds(j * VROW, LANES)]
                for k in range(1, 6):
                    s = s + vals_v[pl.ds(j * VROW + k * LANES, LANES)]
                tail = vals_v[pl.ds(j * VROW + 88, LANES)]
                s = s + jnp.where(tailmask, tail, 0.0)
                for p in perms:
                    s = s + _permute(s, p)
                acc = jnp.where(iota == r, s, acc)
            out_v[pl.ds(g * GRP + blk * LANES, LANES)] = acc

    n_grp = rows_per_w // GRP
    fire(jnp.int32(0))
    fire(jnp.int32(1))

    def pipe(g, _):
        fire(g)
        drain(g - 2)
        reduce_grp(g - 2)
        return 0

    lax.fori_loop(2, n_grp, pipe, 0)
    for g in (n_grp - 2, n_grp - 1):
        drain(jnp.int32(g))
        reduce_grp(jnp.int32(g))

    pltpu.sync_copy(out_v, out_hbm.at[pl.ds(base, rows_per_w)])


def kernel(fids, table):
    B, F = fids.shape
    vocab = table.shape[0]
    rows_per_w = B // NW
    assert B == NW * rows_per_w and F == 100 and rows_per_w % GRP == 0

    fids_p = jnp.pad(fids, ((0, 0), (0, PADF - F)))
    table_flat = table.reshape(vocab)

    mesh = plsc.VectorSubcoreMesh(core_axis_name="c", subcore_axis_name="s")
    run = functools.partial(
        pl.kernel,
        out_type=jax.ShapeDtypeStruct((B,), jnp.float32),
        mesh=mesh,
        scratch_types=[
            pltpu.VMEM((rows_per_w, PADF), jnp.int32),
            pltpu.VMEM((rows_per_w * VROW,), jnp.float32),
            pltpu.VMEM((rows_per_w,), jnp.float32),
            pltpu.SemaphoreType.DMA,
        ],
    )(_lr_kernel)
    return run(fids_p, table_flat)


# speedup vs baseline: 1.0216x; 1.0048x over previous
import functools

import jax
import jax.numpy as jnp
from jax import lax
from jax.experimental import pallas as pl
from jax.experimental.pallas import tpu as pltpu
from jax.experimental.pallas import tpu_sc as plsc

NW = 32
PADF = 128
VROW = 104
GRP = 32
LANES = 16

_GDN = lax.GatherDimensionNumbers(
    offset_dims=(), collapsed_slice_dims=(0,), start_index_map=(0,)
)


def _permute(v, idx):
    return lax.gather(
        v, idx[:, None], _GDN, slice_sizes=(1,),
        mode=lax.GatherScatterMode.PROMISE_IN_BOUNDS,
    )


def _lr_kernel(fids_hbm, table_hbm, out_hbm, idx_v, vals_v, out_v, sem):
    rows_per_w = out_v.shape[0]
    f_per_row = 100

    wid = lax.axis_index("s") * 2 + lax.axis_index("c")
    base = wid * rows_per_w

    pltpu.sync_copy(fids_hbm.at[pl.ds(base, rows_per_w), :], idx_v)

    iota = lax.iota(jnp.int32, LANES)
    perms = [iota ^ s for s in (8, 4, 2, 1)]
    tailmask = (iota >= 8) & (iota < 12)
    zeros16 = jnp.zeros((LANES,), jnp.float32)

    def fire(g):
        for b in range(GRP):
            j = g * GRP + b
            pltpu.async_copy(
                table_hbm.at[idx_v.at[j, pl.ds(0, f_per_row)]],
                vals_v.at[pl.ds(j * VROW, f_per_row)],
                sem,
            )

    def drain(g):
        pltpu.make_async_copy(
            table_hbm.at[pl.ds(0, GRP * f_per_row)],
            vals_v.at[pl.ds(g * GRP * VROW, GRP * f_per_row)],
            sem,
        ).wait()

    def reduce_grp(g):
        for blk in range(GRP // LANES):
            acc = zeros16
            for r in range(LANES):
                j = g * GRP + blk * LANES + r
                s = vals_v[pl.ds(j * VROW, LANES)]
                for k in range(1, 6):
                    s = s + vals_v[pl.ds(j * VROW + k * LANES, LANES)]
                tail = vals_v[pl.ds(j * VROW + 88, LANES)]
                s = s + jnp.where(tailmask, tail, 0.0)
                for p in perms:
                    s = s + _permute(s, p)
                acc = jnp.where(iota == r, s, acc)
            out_v[pl.ds(g * GRP + blk * LANES, LANES)] = acc

    n_grp = rows_per_w // GRP
    fire(jnp.int32(0))
    fire(jnp.int32(1))
    fire(jnp.int32(2))

    def pipe(g, _):
        fire(g)
        drain(g - 3)
        reduce_grp(g - 3)
        return 0

    lax.fori_loop(3, n_grp, pipe, 0)
    for g in (n_grp - 3, n_grp - 2, n_grp - 1):
        drain(jnp.int32(g))
        reduce_grp(jnp.int32(g))

    pltpu.sync_copy(out_v, out_hbm.at[pl.ds(base, rows_per_w)])


def kernel(fids, table):
    B, F = fids.shape
    vocab = table.shape[0]
    rows_per_w = B // NW
    assert B == NW * rows_per_w and F == 100 and rows_per_w % GRP == 0

    fids_p = jnp.pad(fids, ((0, 0), (0, PADF - F)))
    table_flat = table.reshape(vocab)

    mesh = plsc.VectorSubcoreMesh(core_axis_name="c", subcore_axis_name="s")
    run = functools.partial(
        pl.kernel,
        out_type=jax.ShapeDtypeStruct((B,), jnp.float32),
        mesh=mesh,
        scratch_types=[
            pltpu.VMEM((rows_per_w, PADF), jnp.int32),
            pltpu.VMEM((rows_per_w * VROW,), jnp.float32),
            pltpu.VMEM((rows_per_w,), jnp.float32),
            pltpu.SemaphoreType.DMA,
        ],
    )(_lr_kernel)
    return run(fids_p, table_flat)


# speedup vs baseline: 1.0237x; 1.0020x over previous
import functools

import jax
import jax.numpy as jnp
from jax import lax
from jax.experimental import pallas as pl
from jax.experimental.pallas import tpu as pltpu
from jax.experimental.pallas import tpu_sc as plsc

NW = 32
PADF = 128
VROW = 104
GRP = 32
LANES = 16

_GDN = lax.GatherDimensionNumbers(
    offset_dims=(), collapsed_slice_dims=(0,), start_index_map=(0,)
)


def _permute(v, idx):
    return lax.gather(
        v, idx[:, None], _GDN, slice_sizes=(1,),
        mode=lax.GatherScatterMode.PROMISE_IN_BOUNDS,
    )


def _lr_kernel(fids_hbm, table_hbm, out_hbm, idx_v, vals_v, out_v, sem):
    rows_per_w = out_v.shape[0]
    f_per_row = 100

    wid = lax.axis_index("s") * 2 + lax.axis_index("c")
    base = wid * rows_per_w

    pltpu.sync_copy(fids_hbm.at[pl.ds(base, rows_per_w), :], idx_v)

    iota = lax.iota(jnp.int32, LANES)
    perms = [iota ^ s for s in (8, 4, 2, 1)]
    tailmask = (iota >= 8) & (iota < 12)
    zeros16 = jnp.zeros((LANES,), jnp.float32)

    def fire(g):
        for b in range(GRP):
            j = g * GRP + b
            pltpu.async_copy(
                table_hbm.at[idx_v.at[j, pl.ds(0, f_per_row)]],
                vals_v.at[pl.ds(j * VROW, f_per_row)],
                sem,
            )

    def drain(g):
        pltpu.make_async_copy(
            table_hbm.at[pl.ds(0, GRP * f_per_row)],
            vals_v.at[pl.ds(g * GRP * VROW, GRP * f_per_row)],
            sem,
        ).wait()

    def reduce_grp(g):
        for blk in range(GRP // LANES):
            acc = zeros16
            for r in range(LANES):
                j = g * GRP + blk * LANES + r
                s = vals_v[pl.ds(j * VROW, LANES)]
                for k in range(1, 6):
                    s = s + vals_v[pl.ds(j * VROW + k * LANES, LANES)]
                tail = vals_v[pl.ds(j * VROW + 88, LANES)]
                s = s + jnp.where(tailmask, tail, 0.0)
                for p in perms:
                    s = s + _permute(s, p)
                acc = jnp.where(iota == r, s, acc)
            out_v[pl.ds(g * GRP + blk * LANES, LANES)] = acc

    n_grp = rows_per_w // GRP
    for p0 in range(4):
        fire(jnp.int32(p0))

    def pipe(g, _):
        fire(g)
        drain(g - 4)
        reduce_grp(g - 4)
        return 0

    lax.fori_loop(4, n_grp, pipe, 0)
    for g in (n_grp - 4, n_grp - 3, n_grp - 2, n_grp - 1):
        drain(jnp.int32(g))
        reduce_grp(jnp.int32(g))

    pltpu.sync_copy(out_v, out_hbm.at[pl.ds(base, rows_per_w)])


def kernel(fids, table):
    B, F = fids.shape
    vocab = table.shape[0]
    rows_per_w = B // NW
    assert B == NW * rows_per_w and F == 100 and rows_per_w % GRP == 0

    fids_p = jnp.pad(fids, ((0, 0), (0, PADF - F)))
    table_flat = table.reshape(vocab)

    mesh = plsc.VectorSubcoreMesh(core_axis_name="c", subcore_axis_name="s")
    run = functools.partial(
        pl.kernel,
        out_type=jax.ShapeDtypeStruct((B,), jnp.float32),
        mesh=mesh,
        scratch_types=[
            pltpu.VMEM((rows_per_w, PADF), jnp.int32),
            pltpu.VMEM((rows_per_w * VROW,), jnp.float32),
            pltpu.VMEM((rows_per_w,), jnp.float32),
            pltpu.SemaphoreType.DMA,
        ],
    )(_lr_kernel)
    return run(fids_p, table_flat)
